# baseline (speedup 1.0000x reference)
import functools

import jax
import jax.numpy as jnp
from jax import lax
from jax.experimental import pallas as pl
from jax.experimental.pallas import tpu as pltpu
from jax.experimental.pallas import tpu_sc as plsc

B = 16384
SB = 2 * B
D = 64
V = 1000000
L = 16
NC, NS = 2, 16
NW = NC * NS
STEP = 512
NSTEPS = (V + STEP - 1) // STEP
STEPS_Q, STEPS_R = divmod(NSTEPS, NW)
MAXSTEPS = STEPS_Q + 1
LCAP = 2048
BLK = 2048
PAD0 = SB

_mesh = plsc.VectorSubcoreMesh(
    core_axis_name="c", subcore_axis_name="s", num_cores=NC, num_subcores=NS
)

_stage_t = jax.ShapeDtypeStruct((SB + 128, 128), jnp.float32)


@functools.partial(
    pl.kernel,
    out_type=(_stage_t, _stage_t, _stage_t),
    mesh=_mesh,
    scratch_types=[
        pltpu.VMEM((BLK,), jnp.int32),
        pltpu.VMEM((LCAP + L,), jnp.int32),
        pltpu.VMEM((LCAP + L,), jnp.int32),
        pltpu.VMEM((LCAP + L,), jnp.int32),
        pltpu.VMEM((LCAP + L,), jnp.int32),
        pltpu.VMEM((LCAP + L,), jnp.int32),
        pltpu.VMEM((LCAP + L,), jnp.int32),
        pltpu.VMEM((D, STEP), jnp.float32),
        pltpu.VMEM((D, STEP), jnp.float32),
        pltpu.VMEM((D, V - (NSTEPS - 1) * STEP), jnp.float32),
        pltpu.VMEM((256,), jnp.int32),
        pltpu.VMEM((256,), jnp.int32),
        pltpu.VMEM((128, 128), jnp.float32),
        pltpu.VMEM((128,), jnp.int32),
        pltpu.SemaphoreType.DMA,
        pltpu.SemaphoreType.DMA,
        pltpu.SemaphoreType.DMA,
    ],
    compiler_params=pltpu.CompilerParams(
        needs_layout_passes=False, use_tc_tiling_on_sc=True
    ),
)
def _sc_sweep(subj_hbm, rel_hbm, obj_hbm, ent_t, reln_t,
              stage_s, stage_r, stage_o,
              idx_blk, ls_slot, ls_e, lr_slot, lr_e, lo_slot, lo_e,
              buf0, buf1, tailbuf, act_slot, act_le, stage_v, idx_stage,
              sem0, sem1, sem_sc):
    wid = lax.axis_index("s") * NC + lax.axis_index("c")
    s0 = wid * STEPS_Q + jnp.minimum(wid, STEPS_R)
    scount = jnp.where(wid < STEPS_R, STEPS_Q + 1, STEPS_Q)
    lo_ent = s0 * STEP
    hi_ent = jnp.minimum((s0 + scount) * STEP, V)
    iota = lax.iota(jnp.int32, L)

    def route(g_hbm, list_slot, list_e):
        def block(b, cnt):
            off = pl.multiple_of(b * BLK, 8)
            pltpu.sync_copy(g_hbm.at[pl.ds(off, BLK)], idx_blk)

            def chunk(i, cnt):
                e = idx_blk[pl.ds(i * L, L)]
                slotv = b * BLK + i * L + iota
                m = (e >= lo_ent) & (e < hi_ent)
                plsc.store_compressed(list_slot.at[pl.ds(cnt, L)], slotv,
                                      mask=m)
                plsc.store_compressed(list_e.at[pl.ds(cnt, L)], e, mask=m)
                n = plsc.all_reduce_population_count(m)[0]
                return jnp.minimum(cnt + n, jnp.int32(LCAP))

            return lax.fori_loop(0, BLK // L, chunk, cnt)

        return lax.fori_loop(0, SB // BLK, block, jnp.int32(0))

    cnt_s = route(subj_hbm, ls_slot, ls_e)
    cnt_r = route(rel_hbm, lr_slot, lr_e)
    cnt_o = route(obj_hbm, lo_slot, lo_e)

    bufs = (buf0, buf1)
    sems = (sem0, sem1)

    TAIL0 = (NSTEPS - 1) * STEP
    TAILW = V - TAIL0

    def start_step(tab, s, k):
        glo = pl.multiple_of((s0 + s) * STEP, 128)
        pltpu.async_copy(tab.at[:, pl.ds(glo, STEP)], bufs[k], sems[k])

    def wait_step(tab, s, k):
        glo = pl.multiple_of((s0 + s) * STEP, 128)
        pltpu.make_async_copy(tab.at[:, pl.ds(glo, STEP)], bufs[k],
                              sems[k]).wait()

    def process_list(glo, buf, list_slot, list_e, cnt, stage_hbm):
        for p in range(16):
            act_slot[pl.ds(p * L, L)] = jnp.int32(PAD0 + (p % 8) * L) + iota

        def scan_chunk(i, na):
            e = list_e[pl.ds(i * L, L)]
            sl = list_slot[pl.ds(i * L, L)]
            valid = (i * L + iota) < cnt
            m = valid & (e >= glo) & (e < glo + STEP)
            plsc.store_compressed(act_slot.at[pl.ds(na, L)], sl, mask=m)
            plsc.store_compressed(act_le.at[pl.ds(na, L)], e - glo, mask=m)
            n = plsc.all_reduce_population_count(m)[0]
            return jnp.minimum(na + n, jnp.int32(128))

        nchunks = (cnt + (L - 1)) // L
        na = lax.fori_loop(0, nchunks, scan_chunk, jnp.int32(0))

        def extract(j, carry):
            le = act_le[pl.ds(j, L)][0]
            col = jnp.full((L,), 0, jnp.int32) + le
            for kk in range(D // L):
                v = plsc.load_gather(buf, [kk * L + iota, col])
                stage_v[j, pl.ds(kk * L, L)] = v
            return carry

        lax.fori_loop(0, na, extract, 0)

        @pl.when(na > 0)
        def _():
            for p in range(8):
                idx_stage[pl.ds(p * L, L)] = act_slot[pl.ds(p * L, L)]
            pltpu.async_copy(stage_v, stage_hbm.at[idx_stage],
                             sem_sc).wait()

    def sweep(tab, jobs):
        nfull = scount - jnp.where(wid == NW - 1, 1, 0)
        start_step(tab, 0, 0)

        def step_body(s, carry):
            for par in (0, 1):
                @pl.when((s & 1) == par)
                def _():
                    wait_step(tab, s, par)

                    @pl.when(s + 1 < nfull)
                    def _():
                        start_step(tab, s + 1, 1 - par)

                    glo = (s0 + s) * STEP
                    for (list_slot, list_e, cnt, stage_hbm) in jobs:
                        process_list(glo, bufs[par], list_slot, list_e,
                                     cnt, stage_hbm)
            return carry

        lax.fori_loop(0, nfull, step_body, 0)

        @pl.when(wid == NW - 1)
        def _():
            pltpu.sync_copy(tab.at[:, pl.ds(TAIL0, TAILW)], tailbuf)
            for (list_slot, list_e, cnt, stage_hbm) in jobs:
                process_list(jnp.int32(TAIL0), tailbuf, list_slot, list_e,
                             cnt, stage_hbm)

    sweep(ent_t, [(ls_slot, ls_e, cnt_s, stage_s),
                  (lo_slot, lo_e, cnt_o, stage_o)])
    sweep(reln_t, [(lr_slot, lr_e, cnt_r, stage_r)])


def _phaseb_body(a_ref, b_ref, c_ref, o_ref):
    d = (a_ref[:, :D] + b_ref[:, :D] - c_ref[:, :D])
    o_ref[...] = -jnp.sqrt(jnp.sum(d * d, axis=-1))


_PB_ROWS = 512

_phaseb = pl.pallas_call(
    _phaseb_body,
    grid=(SB // _PB_ROWS,),
    in_specs=[pl.BlockSpec((_PB_ROWS, 128), lambda i: (i, 0))] * 3,
    out_specs=pl.BlockSpec((_PB_ROWS,), lambda i: (i,)),
    out_shape=jax.ShapeDtypeStruct((SB,), jnp.float32),
)


def kernel(positive, negative, entity_table, relation_table):
    trip = jnp.concatenate([positive, negative], axis=0)
    subj = trip[:, 0]
    rel = trip[:, 1]
    obj = trip[:, 2]
    stage_s, stage_r, stage_o = _sc_sweep(
        subj, rel, obj, entity_table.T, relation_table.T)
    scores = _phaseb(stage_s, stage_r, stage_o)
    return scores[:B], scores[B:]


# speedup vs baseline: 1.0260x; 1.0260x over previous
import functools

import jax
import jax.numpy as jnp
from jax import lax
from jax.experimental import pallas as pl
from jax.experimental.pallas import tpu as pltpu
from jax.experimental.pallas import tpu_sc as plsc

B = 16384
SB = 2 * B
D = 64
V = 1000000
L = 16
NC, NS = 2, 16
NW = NC * NS
STEP = 512
NSTEPS = (V + STEP - 1) // STEP
STEPS_Q, STEPS_R = divmod(NSTEPS, NW)
LCAP = 2048
BLK = 2048
PAD0 = SB
NB = 8
BCAP = 320
BSH = 12

_mesh = plsc.VectorSubcoreMesh(
    core_axis_name="c", subcore_axis_name="s", num_cores=NC, num_subcores=NS
)

_stage_t = jax.ShapeDtypeStruct((SB + 128, 128), jnp.float32)


@functools.partial(
    pl.kernel,
    out_type=(_stage_t, _stage_t, _stage_t),
    mesh=_mesh,
    scratch_types=[
        pltpu.VMEM((BLK,), jnp.int32),
        pltpu.VMEM((LCAP + L,), jnp.int32),
        pltpu.VMEM((LCAP + L,), jnp.int32),
        pltpu.VMEM((NB * BCAP + L,), jnp.int32),
        pltpu.VMEM((NB * BCAP + L,), jnp.int32),
        pltpu.VMEM((NB * BCAP + L,), jnp.int32),
        pltpu.VMEM((NB * BCAP + L,), jnp.int32),
        pltpu.VMEM((NB * BCAP + L,), jnp.int32),
        pltpu.VMEM((NB * BCAP + L,), jnp.int32),
        pltpu.VMEM((L,), jnp.int32),
        pltpu.VMEM((L,), jnp.int32),
        pltpu.VMEM((L,), jnp.int32),
        pltpu.VMEM((D, STEP), jnp.float32),
        pltpu.VMEM((D, STEP), jnp.float32),
        pltpu.VMEM((D, V - (NSTEPS - 1) * STEP), jnp.float32),
        pltpu.VMEM((192,), jnp.int32),
        pltpu.VMEM((192,), jnp.int32),
        pltpu.VMEM((128, 128), jnp.float32),
        pltpu.VMEM((128, 128), jnp.float32),
        pltpu.VMEM((128,), jnp.int32),
        pltpu.VMEM((128,), jnp.int32),
        pltpu.SemaphoreType.DMA,
        pltpu.SemaphoreType.DMA,
        pltpu.SemaphoreType.DMA,
        pltpu.SemaphoreType.DMA,
    ],
    compiler_params=pltpu.CompilerParams(
        needs_layout_passes=False, use_tc_tiling_on_sc=True
    ),
)
def _sc_sweep(subj_hbm, rel_hbm, obj_hbm, ent_t, reln_t,
              stage_s, stage_r, stage_o,
              idx_blk, fl_slot, fl_e,
              bs_slot, bs_e, br_slot, br_e, bo_slot, bo_e,
              cs_v, cr_v, co_v,
              buf0, buf1, tailbuf, act_slot, act_le,
              stage_v0, stage_v1, idx_st0, idx_st1,
              sem0, sem1, sem_sc0, sem_sc1):
    wid = lax.axis_index("s") * NC + lax.axis_index("c")
    s0 = wid * STEPS_Q + jnp.minimum(wid, STEPS_R)
    scount = jnp.where(wid < STEPS_R, STEPS_Q + 1, STEPS_Q)
    lo_ent = s0 * STEP
    hi_ent = jnp.minimum((s0 + scount) * STEP, V)
    iota = lax.iota(jnp.int32, L)
    lane0 = iota == 0

    def route_and_bucket(g_hbm, bk_slot, bk_e, cnt_v):
        def block(b, cnt):
            off = pl.multiple_of(b * BLK, 8)
            pltpu.sync_copy(g_hbm.at[pl.ds(off, BLK)], idx_blk)

            def chunk(i, cnt):
                e = idx_blk[pl.ds(i * L, L)]
                slotv = b * BLK + i * L + iota
                m = (e >= lo_ent) & (e < hi_ent)
                plsc.store_compressed(fl_slot.at[pl.ds(cnt, L)], slotv,
                                      mask=m)
                plsc.store_compressed(fl_e.at[pl.ds(cnt, L)], e, mask=m)
                return cnt + plsc.all_reduce_population_count(m)[0]

            return lax.fori_loop(0, BLK // L, chunk, cnt)

        cnt = lax.fori_loop(0, SB // BLK, block, jnp.int32(0))

        def part_chunk(i, cnts):
            e = fl_e[pl.ds(i * L, L)]
            sl = fl_slot[pl.ds(i * L, L)]
            valid = (i * L + iota) < cnt
            cb = lax.shift_right_logical(e - lo_ent, BSH)
            new = []
            for k in range(NB):
                m = valid & (cb == k)
                plsc.store_compressed(
                    bk_slot.at[pl.ds(k * BCAP + cnts[k], L)], sl, mask=m)
                plsc.store_compressed(
                    bk_e.at[pl.ds(k * BCAP + cnts[k], L)], e, mask=m)
                new.append(cnts[k]
                           + plsc.all_reduce_population_count(m)[0])
            return tuple(new)

        nchunks = (cnt + (L - 1)) // L
        cnts = lax.fori_loop(0, nchunks, part_chunk,
                             tuple(jnp.int32(0) for _ in range(NB)))
        for k in range(NB):
            plsc.store_scatter(cnt_v, [jnp.full((L,), 0, jnp.int32) + k],
                               jnp.full((L,), 0, jnp.int32) + cnts[k],
                               mask=lane0)

    with jax.named_scope("routing"):
        route_and_bucket(subj_hbm, bs_slot, bs_e, cs_v)
        route_and_bucket(rel_hbm, br_slot, br_e, cr_v)
        route_and_bucket(obj_hbm, bo_slot, bo_e, co_v)

    bufs = (buf0, buf1)
    sems = (sem0, sem1)
    stages_v = (stage_v0, stage_v1)
    idx_sts = (idx_st0, idx_st1)
    sems_sc = (sem_sc0, sem_sc1)

    TAIL0 = (NSTEPS - 1) * STEP
    TAILW = V - TAIL0

    def start_step(tab, s, k):
        glo = pl.multiple_of((s0 + s) * STEP, 128)
        pltpu.async_copy(tab.at[:, pl.ds(glo, STEP)], bufs[k], sems[k])

    def wait_step(tab, s, k):
        glo = pl.multiple_of((s0 + s) * STEP, 128)
        pltpu.make_async_copy(tab.at[:, pl.ds(glo, STEP)], bufs[k],
                              sems[k]).wait()

    def wait_scatter(sp, prev_hbm):
        pltpu.make_async_copy(stages_v[sp], prev_hbm.at[idx_sts[sp]],
                              sems_sc[sp]).wait()

    def process_list(s, glo, buf, sp, job, prev_hbm, wait_pred):
        bk_slot, bk_e, cnt_v, stage_hbm = job
        for p in range(12):
            act_slot[pl.ds(p * L, L)] = jnp.int32(PAD0 + (p % 8) * L) + iota

        cb = lax.shift_right_logical(jnp.int32((s0 + s) * STEP) - lo_ent,
                                     BSH)
        base = cb * BCAP
        cnt = plsc.load_gather(cnt_v, [jnp.full((L,), 0, jnp.int32)
                                       + cb])[0]

        def scan_chunk(i, na):
            e = bk_e[pl.ds(base + i * L, L)]
            sl = bk_slot[pl.ds(base + i * L, L)]
            valid = (i * L + iota) < cnt
            m = valid & (e >= glo) & (e < glo + STEP)
            plsc.store_compressed(act_slot.at[pl.ds(na, L)], sl, mask=m)
            plsc.store_compressed(act_le.at[pl.ds(na, L)], e - glo, mask=m)
            return na + plsc.all_reduce_population_count(m)[0]

        nchunks = (cnt + (L - 1)) // L
        na = lax.fori_loop(0, nchunks, scan_chunk, jnp.int32(0))

        @pl.when(wait_pred)
        def _():
            wait_scatter(sp, prev_hbm)

        stage_v = stages_v[sp]
        idx_st = idx_sts[sp]

        def extract(j, carry):
            le = act_le[pl.ds(j, L)][0]
            col = jnp.full((L,), 0, jnp.int32) + le
            for kk in range(D // L):
                v = plsc.load_gather(buf, [kk * L + iota, col])
                stage_v[j, pl.ds(kk * L, L)] = v
            return carry

        lax.fori_loop(0, na, extract, 0)

        for p in range(8):
            idx_st[pl.ds(p * L, L)] = act_slot[pl.ds(p * L, L)]
        pltpu.async_copy(stage_v, stage_hbm.at[idx_st], sems_sc[sp])

    def sweep(tab, jobs):
        nfull = scount - jnp.where(wid == NW - 1, 1, 0)
        start_step(tab, 0, 0)

        def step_body(s, carry):
            for par in (0, 1):
                @pl.when((s & 1) == par)
                def _():
                    wait_step(tab, s, par)

                    @pl.when(s + 1 < nfull)
                    def _():
                        start_step(tab, s + 1, 1 - par)

                    glo = (s0 + s) * STEP
                    for j, job in enumerate(jobs):
                        sp = (par + j) % 2
                        if len(jobs) == 2:
                            prev = jobs[1 - j][3]
                            pred = s >= 1
                        else:
                            prev = job[3]
                            pred = s >= 2
                        process_list(s, glo, bufs[par], sp, job, prev,
                                     pred)
            return carry

        lax.fori_loop(0, nfull, step_body, 0)

        if len(jobs) == 2:
            for par in (0, 1):
                @pl.when(((nfull - 1) & 1) == par)
                def _():
                    for j, job in enumerate(jobs):
                        wait_scatter((par + j) % 2, job[3])
        else:
            for par in (0, 1):
                @pl.when(((nfull - 1) & 1) == par)
                def _():
                    wait_scatter(par, jobs[0][3])

                    @pl.when(nfull >= 2)
                    def _():
                        wait_scatter(1 - par, jobs[0][3])

        @pl.when(wid == NW - 1)
        def _():
            pltpu.sync_copy(tab.at[:, pl.ds(TAIL0, TAILW)], tailbuf)
            for j, job in enumerate(jobs):
                process_list(scount - 1, jnp.int32(TAIL0), tailbuf, j % 2,
                             job, job[3], jnp.bool_(False))
                wait_scatter(j % 2, job[3])

    with jax.named_scope("sweep_ent"):
        sweep(ent_t, [(bs_slot, bs_e, cs_v, stage_s),
                      (bo_slot, bo_e, co_v, stage_o)])
    with jax.named_scope("sweep_rel"):
        sweep(reln_t, [(br_slot, br_e, cr_v, stage_r)])


def _phaseb_body(a_ref, b_ref, c_ref, o_ref):
    d = (a_ref[:, :D] + b_ref[:, :D] - c_ref[:, :D])
    o_ref[...] = -jnp.sqrt(jnp.sum(d * d, axis=-1))


_PB_ROWS = 2048

_phaseb = pl.pallas_call(
    _phaseb_body,
    grid=(SB // _PB_ROWS,),
    in_specs=[pl.BlockSpec((_PB_ROWS, 128), lambda i: (i, 0))] * 3,
    out_specs=pl.BlockSpec((_PB_ROWS,), lambda i: (i,)),
    out_shape=jax.ShapeDtypeStruct((SB,), jnp.float32),
    compiler_params=pltpu.CompilerParams(
        dimension_semantics=("arbitrary",)),
)


def kernel(positive, negative, entity_table, relation_table):
    trip = jnp.concatenate([positive, negative], axis=0)
    subj = trip[:, 0]
    rel = trip[:, 1]
    obj = trip[:, 2]
    stage_s, stage_r, stage_o = _sc_sweep(
        subj, rel, obj, entity_table.T, relation_table.T)
    scores = _phaseb(stage_s, stage_r, stage_o)
    return scores[:B], scores[B:]
